# baseline (speedup 1.0000x reference)
import jax
import jax.numpy as jnp
from jax import lax
from jax.experimental import pallas as pl
from jax.experimental.pallas import tpu as pltpu
from jax.experimental.pallas import tpu_sc as plsc

N = 10000
E = 160000
D = 256
G = 32

NC, NS, L = 2, 16, 16
NW = NC * NS
EK = 128
EPAD = 163840
ROWS_E = EPAD // EK
NPT = 320
ACC_R = 336
DUMP = 320
CAP = 8192
GK = 64
SUP = 32
DEGS = 384
CL = 128

RB = 2000
GRID = N // RB

_mesh = plsc.VectorSubcoreMesh(
    core_axis_name="c", subcore_axis_name="s", num_cores=NC, num_subcores=NS
)

_SCAN_SUP = 64
_N_SCAN = ROWS_E // _SCAN_SUP


def _route_body(src_hbm, dst_hbm, deg_hbm, lsrc_hbm, ldst_hbm, cnt_hbm,
                src_sv, dst_sv, lsrc_v, ldst_v, hist_v, cnt_v):
    c = lax.axis_index("c")
    s = lax.axis_index("s")
    w = c * NS + s
    base = w * NPT

    zero16f = jnp.zeros((L,), jnp.float32)
    zero16i = jnp.zeros((L,), jnp.int32)
    dump16 = jnp.full((L,), DUMP, jnp.int32)
    ones16 = jnp.ones((L,), jnp.float32)

    @pl.loop(0, DEGS // L)
    def _zero_hist(u):
        hist_v[pl.ds(u * L, L)] = zero16f

    @pl.loop(0, CAP // L)
    def _prefill(u):
        lsrc_v[pl.ds(u * L, L)] = zero16i
        ldst_v[pl.ds(u * L, L)] = dump16

    def _scan_sup(t, cnt):
        pltpu.sync_copy(src_hbm.at[pl.ds(t * _SCAN_SUP, _SCAN_SUP)], src_sv)
        pltpu.sync_copy(dst_hbm.at[pl.ds(t * _SCAN_SUP, _SCAN_SUP)], dst_sv)

        def _group(u, cnt):
            r = u // (EK // L)
            col = (u % (EK // L)) * L
            d = dst_sv[r, pl.ds(col, L)]
            sv = src_sv[r, pl.ds(col, L)]
            own = lax.shift_right_logical(d * 52429, 24)
            lidx = d - base
            m = own == w
            safe = cnt < CAP - 2 * L
            msafe = m & safe
            plsc.addupdate_scatter(hist_v, [lidx], ones16, mask=msafe)
            plsc.store_compressed(lsrc_v.at[pl.ds(cnt, L)], sv, mask=msafe)
            plsc.store_compressed(ldst_v.at[pl.ds(cnt, L)], lidx, mask=msafe)
            return cnt + jnp.sum(msafe.astype(jnp.int32))

        return lax.fori_loop(0, _SCAN_SUP * EK // L, _group, cnt)

    cnt = lax.fori_loop(0, _N_SCAN, _scan_sup, jnp.int32(0))

    cnt16 = jnp.broadcast_to(cnt, (L,)).astype(jnp.int32)

    @pl.loop(0, CL // L)
    def _fill_cnt(u):
        cnt_v[pl.ds(u * L, L)] = cnt16

    pltpu.sync_copy(hist_v, deg_hbm.at[pl.ds(w * DEGS, DEGS)])
    pltpu.sync_copy(lsrc_v, lsrc_hbm.at[pl.ds(w * CAP, CAP)])
    pltpu.sync_copy(ldst_v, ldst_hbm.at[pl.ds(w * CAP, CAP)])
    pltpu.sync_copy(cnt_v, cnt_hbm.at[pl.ds(w * CL, CL)])


_route_call = pl.kernel(
    _route_body,
    out_type=(
        jax.ShapeDtypeStruct((NW * DEGS,), jnp.float32),
        jax.ShapeDtypeStruct((NW * CAP,), jnp.int32),
        jax.ShapeDtypeStruct((NW * CAP,), jnp.int32),
        jax.ShapeDtypeStruct((NW * CL,), jnp.int32),
    ),
    mesh=_mesh,
    compiler_params=pltpu.CompilerParams(needs_layout_passes=False),
    scratch_types=[
        pltpu.VMEM((_SCAN_SUP, EK), jnp.int32),
        pltpu.VMEM((_SCAN_SUP, EK), jnp.int32),
        pltpu.VMEM((CAP,), jnp.int32),
        pltpu.VMEM((CAP,), jnp.int32),
        pltpu.VMEM((DEGS,), jnp.float32),
        pltpu.VMEM((CL,), jnp.int32),
    ],
)


def _agg_body(h_hbm, lsrc_hbm, ldst_hbm, cnt_hbm, zeros_hbm, out_hbm,
              lsrc_v, ldst_v, cnt_v, rows0, rows1, sem0, sem1, acc):
    c = lax.axis_index("c")
    s = lax.axis_index("s")
    w = c * NS + s

    pltpu.sync_copy(cnt_hbm.at[pl.ds(w * CL, CL)], cnt_v)
    cnt = cnt_v[pl.ds(0, L)][0]
    n_ch = lax.div(cnt + (GK - 1), GK)
    n_sup = lax.div(n_ch + (SUP - 1), SUP)

    @pl.loop(0, ACC_R // 16)
    def _zero(k):
        pltpu.sync_copy(zeros_hbm, acc.at[pl.ds(k * 16, 16)])

    bufs = (rows0, rows1)
    sems = (sem0, sem1)

    def _super(t, _):
        pltpu.sync_copy(lsrc_hbm.at[pl.ds(w * CAP + t * (SUP * GK), SUP * GK)],
                        lsrc_v)
        pltpu.sync_copy(ldst_hbm.at[pl.ds(w * CAP + t * (SUP * GK), SUP * GK)],
                        ldst_v)
        ch0 = t * SUP

        @pl.when(ch0 + 0 < n_ch)
        def _():
            pltpu.async_copy(h_hbm.at[lsrc_v.at[pl.ds(0, GK)]], rows0, sem0)

        @pl.when(ch0 + 1 < n_ch)
        def _():
            pltpu.async_copy(h_hbm.at[lsrc_v.at[pl.ds(GK, GK)]], rows1, sem1)

        @pl.loop(0, SUP, step=2)
        def _chunk(k):
            for b in range(2):
                kk = k + b
                buf = bufs[b]
                sem = sems[b]

                @pl.when(ch0 + kk < n_ch)
                def _():
                    pltpu.make_async_copy(
                        h_hbm.at[lsrc_v.at[pl.ds(kk * GK, GK)]], buf,
                        sem).wait()

                    @pl.loop(0, GK // L)
                    def _edges(gidx):
                        ld16 = ldst_v[pl.ds(kk * GK + gidx * L, L)]
                        for e2 in range(L):
                            ld = ld16[e2]
                            e = gidx * L + e2
                            for seg in range(D // L):
                                plsc.addupdate(
                                    acc.at[ld, pl.ds(seg * L, L)],
                                    buf[e, pl.ds(seg * L, L)])

                    @pl.when((kk + 2 < SUP) & (ch0 + kk + 2 < n_ch))
                    def _():
                        pltpu.async_copy(
                            h_hbm.at[lsrc_v.at[pl.ds((kk + 2) * GK, GK)]],
                            buf, sem)

        return ()

    lax.fori_loop(0, n_sup, _super, ())

    @pl.loop(0, NPT // 16)
    def _writeout(k):
        pltpu.sync_copy(acc.at[pl.ds(k * 16, 16)],
                        out_hbm.at[pl.ds(w * NPT + k * 16, 16)])


_agg_call = pl.kernel(
    _agg_body,
    out_type=jax.ShapeDtypeStruct((NW * NPT, D), jnp.float32),
    mesh=_mesh,
    compiler_params=pltpu.CompilerParams(needs_layout_passes=False),
    scratch_types=[
        pltpu.VMEM((SUP * GK,), jnp.int32),
        pltpu.VMEM((SUP * GK,), jnp.int32),
        pltpu.VMEM((CL,), jnp.int32),
        pltpu.VMEM((GK, D), jnp.float32),
        pltpu.VMEM((GK, D), jnp.float32),
        pltpu.SemaphoreType.DMA,
        pltpu.SemaphoreType.DMA,
        pltpu.VMEM((ACC_R, D), jnp.float32),
    ],
)


def _mm1_kernel(x_ref, w_ref, d_ref, h_ref, hp_ref, dinv_ref):
    dinv = lax.rsqrt(d_ref[...] + 1.0)
    h = jnp.dot(x_ref[...], w_ref[...], preferred_element_type=jnp.float32)
    h_ref[...] = h
    hp_ref[...] = h * dinv
    dinv_ref[...] = dinv


def _mm2_kernel(a_ref, h1_ref, dinv_ref, b_ref, w_ref, h2_ref, h2p_ref):
    di = dinv_ref[...]
    z = jnp.maximum(di * a_ref[...] + di * di * h1_ref[...] + b_ref[...], 0.0)
    h2 = jnp.dot(z, w_ref[...], preferred_element_type=jnp.float32)
    h2_ref[...] = h2
    h2p_ref[...] = h2 * di


def _final_kernel(a_ref, h2_ref, dinv_ref, b_ref, batch_ref,
                  wc1_ref, bc1_ref, wc2_ref, bc2_ref, wc3_ref, bc3_ref,
                  out_ref, gacc, cacc):
    i = pl.program_id(0)

    @pl.when(i == 0)
    def _():
        gacc[...] = jnp.zeros((G, D), jnp.float32)
        cacc[...] = jnp.zeros((G, 1), jnp.float32)

    di = dinv_ref[...]
    z = jnp.maximum(di * a_ref[...] + di * di * h2_ref[...] + b_ref[...], 0.0)
    bvec = batch_ref[...]
    oh = (bvec == lax.broadcasted_iota(jnp.int32, (RB, G), 1)).astype(
        jnp.float32)
    gacc[...] += lax.dot_general(oh, z, (((0,), (0,)), ((), ())),
                                 preferred_element_type=jnp.float32)
    cacc[...] += lax.dot_general(oh, jnp.ones((RB, 1), jnp.float32),
                                 (((0,), (0,)), ((), ())),
                                 preferred_element_type=jnp.float32)

    @pl.when(i == GRID - 1)
    def _():
        g = gacc[...] / jnp.maximum(cacc[...], 1.0)
        g = jnp.maximum(
            jnp.dot(g, wc1_ref[...], preferred_element_type=jnp.float32)
            + bc1_ref[...], 0.0)
        g = jnp.maximum(
            jnp.dot(g, wc2_ref[...], preferred_element_type=jnp.float32)
            + bc2_ref[...], 0.0)
        out_ref[...] = (jnp.dot(g, wc3_ref[...],
                                preferred_element_type=jnp.float32)
                        + bc3_ref[...])


def _row_spec():
    return pl.BlockSpec((RB, D), lambda i: (i, 0))


def _col_spec():
    return pl.BlockSpec((RB, 1), lambda i: (i, 0))


def _full_spec(r, c):
    return pl.BlockSpec((r, c), lambda i: (0, 0))


def kernel(x, edge_index, batch, W1, b1, W2, b2, Wc1, bc1, Wc2, bc2, Wc3, bc3):
    src = edge_index[0].astype(jnp.int32)
    dst = edge_index[1].astype(jnp.int32)
    src2 = jnp.concatenate(
        [src, jnp.zeros((EPAD - E,), jnp.int32)]).reshape(ROWS_E, EK)
    dst2 = jnp.concatenate(
        [dst, jnp.full((EPAD - E,), N, jnp.int32)]).reshape(ROWS_E, EK)
    batch2 = batch.astype(jnp.int32).reshape(N, 1)

    zeros_r = jnp.zeros((16, D), jnp.float32)

    dego, lsrc, ldst, cnts = _route_call(src2, dst2)
    d = dego.reshape(NW, DEGS)[:, :NPT].reshape(NW * NPT)[:N].reshape(N, 1)

    h1, h1p, dinv = pl.pallas_call(
        _mm1_kernel,
        grid=(GRID,),
        in_specs=[_row_spec(), _full_spec(D, D), _col_spec()],
        out_specs=(_row_spec(), _row_spec(), _col_spec()),
        out_shape=(
            jax.ShapeDtypeStruct((N, D), jnp.float32),
            jax.ShapeDtypeStruct((N, D), jnp.float32),
            jax.ShapeDtypeStruct((N, 1), jnp.float32),
        ),
    )(x, W1, d)

    aggo1 = _agg_call(h1p, lsrc, ldst, cnts, zeros_r)
    a1 = aggo1[:N]

    h2, h2p = pl.pallas_call(
        _mm2_kernel,
        grid=(GRID,),
        in_specs=[_row_spec(), _row_spec(), _col_spec(),
                  _full_spec(1, D), _full_spec(D, D)],
        out_specs=(_row_spec(), _row_spec()),
        out_shape=(
            jax.ShapeDtypeStruct((N, D), jnp.float32),
            jax.ShapeDtypeStruct((N, D), jnp.float32),
        ),
    )(a1, h1, dinv, b1.reshape(1, D), W2)

    aggo2 = _agg_call(h2p, lsrc, ldst, cnts, zeros_r)
    a2 = aggo2[:N]

    out = pl.pallas_call(
        _final_kernel,
        grid=(GRID,),
        in_specs=[_row_spec(), _row_spec(), _col_spec(),
                  _full_spec(1, D), _col_spec(),
                  _full_spec(D, 128), _full_spec(1, 128),
                  _full_spec(128, 64), _full_spec(1, 64),
                  _full_spec(64, 1), _full_spec(1, 1)],
        out_specs=pl.BlockSpec((G, 1), lambda i: (0, 0)),
        out_shape=jax.ShapeDtypeStruct((G, 1), jnp.float32),
        scratch_shapes=[
            pltpu.VMEM((G, D), jnp.float32),
            pltpu.VMEM((G, 1), jnp.float32),
        ],
    )(a2, h2, dinv, b2.reshape(1, D), batch2,
      Wc1, bc1.reshape(1, 128), Wc2, bc2.reshape(1, 64),
      Wc3, bc3.reshape(1, 1))
    return out


# speedup vs baseline: 1.1006x; 1.1006x over previous
import jax
import jax.numpy as jnp
from jax import lax
from jax.experimental import pallas as pl
from jax.experimental.pallas import tpu as pltpu
from jax.experimental.pallas import tpu_sc as plsc

N = 10000
E = 160000
D = 256
G = 32

NC, NS, L = 2, 16, 16
NW = NC * NS
EK = 128
EPAD = 163840
ROWS_E = EPAD // EK
NPT = 320
ACC_R = 336
DUMP = 320
CAP = 8192
GK = 64
SUP = 32
DEGS = 384
CL = 128

RB = 2000
GRID = N // RB

_mesh = plsc.VectorSubcoreMesh(
    core_axis_name="c", subcore_axis_name="s", num_cores=NC, num_subcores=NS
)

_SCAN_SUP = 64
_N_SCAN = ROWS_E // _SCAN_SUP


def _route_body(src_hbm, dst_hbm, deg_hbm, lsrc_hbm, ldst_hbm, cnt_hbm,
                src_sv, dst_sv, lsrc_v, ldst_v, hist_v, cnt_v):
    c = lax.axis_index("c")
    s = lax.axis_index("s")
    w = c * NS + s
    base = w * NPT

    zero16f = jnp.zeros((L,), jnp.float32)
    zero16i = jnp.zeros((L,), jnp.int32)
    dump16 = jnp.full((L,), DUMP, jnp.int32)
    ones16 = jnp.ones((L,), jnp.float32)

    @pl.loop(0, DEGS // L)
    def _zero_hist(u):
        hist_v[pl.ds(u * L, L)] = zero16f

    @pl.loop(0, CAP // L)
    def _prefill(u):
        lsrc_v[pl.ds(u * L, L)] = zero16i
        ldst_v[pl.ds(u * L, L)] = dump16

    def _scan_sup(t, cnt):
        pltpu.sync_copy(src_hbm.at[pl.ds(t * _SCAN_SUP, _SCAN_SUP)], src_sv)
        pltpu.sync_copy(dst_hbm.at[pl.ds(t * _SCAN_SUP, _SCAN_SUP)], dst_sv)

        def _group(u, cnt):
            r = u // (EK // L)
            col = (u % (EK // L)) * L
            d = dst_sv[r, pl.ds(col, L)]
            sv = src_sv[r, pl.ds(col, L)]
            own = lax.shift_right_logical(d * 52429, 24)
            lidx = d - base
            m = own == w
            safe = cnt < CAP - 2 * L
            msafe = m & safe
            plsc.addupdate_scatter(hist_v, [lidx], ones16, mask=msafe)
            plsc.store_compressed(lsrc_v.at[pl.ds(cnt, L)], sv, mask=msafe)
            plsc.store_compressed(ldst_v.at[pl.ds(cnt, L)], lidx, mask=msafe)
            return cnt + jnp.sum(msafe.astype(jnp.int32))

        return lax.fori_loop(0, _SCAN_SUP * EK // L, _group, cnt)

    cnt = lax.fori_loop(0, _N_SCAN, _scan_sup, jnp.int32(0))

    cnt16 = jnp.broadcast_to(cnt, (L,)).astype(jnp.int32)

    @pl.loop(0, CL // L)
    def _fill_cnt(u):
        cnt_v[pl.ds(u * L, L)] = cnt16

    pltpu.sync_copy(hist_v, deg_hbm.at[pl.ds(w * DEGS, DEGS)])
    pltpu.sync_copy(lsrc_v, lsrc_hbm.at[pl.ds(w * CAP, CAP)])
    pltpu.sync_copy(ldst_v, ldst_hbm.at[pl.ds(w * CAP, CAP)])
    pltpu.sync_copy(cnt_v, cnt_hbm.at[pl.ds(w * CL, CL)])


_route_call = pl.kernel(
    _route_body,
    out_type=(
        jax.ShapeDtypeStruct((NW * DEGS,), jnp.float32),
        jax.ShapeDtypeStruct((NW * CAP,), jnp.int32),
        jax.ShapeDtypeStruct((NW * CAP,), jnp.int32),
        jax.ShapeDtypeStruct((NW * CL,), jnp.int32),
    ),
    mesh=_mesh,
    compiler_params=pltpu.CompilerParams(needs_layout_passes=False),
    scratch_types=[
        pltpu.VMEM((_SCAN_SUP, EK), jnp.int32),
        pltpu.VMEM((_SCAN_SUP, EK), jnp.int32),
        pltpu.VMEM((CAP,), jnp.int32),
        pltpu.VMEM((CAP,), jnp.int32),
        pltpu.VMEM((DEGS,), jnp.float32),
        pltpu.VMEM((CL,), jnp.int32),
    ],
)


def _agg_body(h_hbm, lsrc_hbm, ldst_hbm, cnt_hbm, zeros_hbm, out_hbm,
              lsrc_v, ldst_v, cnt_v, rows0, rows1, sem0, sem1, acc):
    c = lax.axis_index("c")
    s = lax.axis_index("s")
    w = c * NS + s

    pltpu.sync_copy(cnt_hbm.at[pl.ds(w * CL, CL)], cnt_v)
    cnt = cnt_v[pl.ds(0, L)][0]
    n_ch = lax.div(cnt + (GK - 1), GK)
    n_sup = lax.div(n_ch + (SUP - 1), SUP)

    @pl.loop(0, ACC_R // 16)
    def _zero(k):
        pltpu.sync_copy(zeros_hbm, acc.at[pl.ds(k * 16, 16)])

    bufs = (rows0, rows1)
    sems = (sem0, sem1)

    def _super(t, _):
        pltpu.sync_copy(lsrc_hbm.at[pl.ds(w * CAP + t * (SUP * GK), SUP * GK)],
                        lsrc_v)
        pltpu.sync_copy(ldst_hbm.at[pl.ds(w * CAP + t * (SUP * GK), SUP * GK)],
                        ldst_v)
        ch0 = t * SUP

        @pl.when(ch0 + 0 < n_ch)
        def _():
            pltpu.async_copy(h_hbm.at[lsrc_v.at[pl.ds(0, GK)]], rows0, sem0)

        @pl.when(ch0 + 1 < n_ch)
        def _():
            pltpu.async_copy(h_hbm.at[lsrc_v.at[pl.ds(GK, GK)]], rows1, sem1)

        @pl.loop(0, SUP, step=2)
        def _chunk(k):
            for b in range(2):
                kk = k + b
                buf = bufs[b]
                sem = sems[b]

                @pl.when(ch0 + kk < n_ch)
                def _():
                    pltpu.make_async_copy(
                        h_hbm.at[lsrc_v.at[pl.ds(kk * GK, GK)]], buf,
                        sem).wait()

                    @pl.loop(0, GK // L)
                    def _edges(gidx):
                        ld16 = ldst_v[pl.ds(kk * GK + gidx * L, L)]
                        for e2 in range(L):
                            ld = ld16[e2]
                            e = gidx * L + e2
                            vals = [buf[e, pl.ds(seg * L, L)]
                                    for seg in range(D // L)]
                            for seg in range(D // L):
                                plsc.addupdate(
                                    acc.at[ld, pl.ds(seg * L, L)],
                                    vals[seg])

                    @pl.when((kk + 2 < SUP) & (ch0 + kk + 2 < n_ch))
                    def _():
                        pltpu.async_copy(
                            h_hbm.at[lsrc_v.at[pl.ds((kk + 2) * GK, GK)]],
                            buf, sem)

        return ()

    lax.fori_loop(0, n_sup, _super, ())

    @pl.loop(0, NPT // 16)
    def _writeout(k):
        pltpu.sync_copy(acc.at[pl.ds(k * 16, 16)],
                        out_hbm.at[pl.ds(w * NPT + k * 16, 16)])


_agg_call = pl.kernel(
    _agg_body,
    out_type=jax.ShapeDtypeStruct((NW * NPT, D), jnp.float32),
    mesh=_mesh,
    compiler_params=pltpu.CompilerParams(needs_layout_passes=False),
    scratch_types=[
        pltpu.VMEM((SUP * GK,), jnp.int32),
        pltpu.VMEM((SUP * GK,), jnp.int32),
        pltpu.VMEM((CL,), jnp.int32),
        pltpu.VMEM((GK, D), jnp.float32),
        pltpu.VMEM((GK, D), jnp.float32),
        pltpu.SemaphoreType.DMA,
        pltpu.SemaphoreType.DMA,
        pltpu.VMEM((ACC_R, D), jnp.float32),
    ],
)


def _mm1_kernel(x_ref, w_ref, d_ref, h_ref, hp_ref, dinv_ref):
    dinv = lax.rsqrt(d_ref[...] + 1.0)
    h = jnp.dot(x_ref[...], w_ref[...], preferred_element_type=jnp.float32)
    h_ref[...] = h
    hp_ref[...] = h * dinv
    dinv_ref[...] = dinv


def _mm2_kernel(a_ref, h1_ref, dinv_ref, b_ref, w_ref, h2_ref, h2p_ref):
    di = dinv_ref[...]
    z = jnp.maximum(di * a_ref[...] + di * di * h1_ref[...] + b_ref[...], 0.0)
    h2 = jnp.dot(z, w_ref[...], preferred_element_type=jnp.float32)
    h2_ref[...] = h2
    h2p_ref[...] = h2 * di


def _final_kernel(a_ref, h2_ref, dinv_ref, b_ref, batch_ref,
                  wc1_ref, bc1_ref, wc2_ref, bc2_ref, wc3_ref, bc3_ref,
                  out_ref, gacc, cacc):
    i = pl.program_id(0)

    @pl.when(i == 0)
    def _():
        gacc[...] = jnp.zeros((G, D), jnp.float32)
        cacc[...] = jnp.zeros((G, 1), jnp.float32)

    di = dinv_ref[...]
    z = jnp.maximum(di * a_ref[...] + di * di * h2_ref[...] + b_ref[...], 0.0)
    bvec = batch_ref[...]
    oh = (bvec == lax.broadcasted_iota(jnp.int32, (RB, G), 1)).astype(
        jnp.float32)
    gacc[...] += lax.dot_general(oh, z, (((0,), (0,)), ((), ())),
                                 preferred_element_type=jnp.float32)
    cacc[...] += lax.dot_general(oh, jnp.ones((RB, 1), jnp.float32),
                                 (((0,), (0,)), ((), ())),
                                 preferred_element_type=jnp.float32)

    @pl.when(i == GRID - 1)
    def _():
        g = gacc[...] / jnp.maximum(cacc[...], 1.0)
        g = jnp.maximum(
            jnp.dot(g, wc1_ref[...], preferred_element_type=jnp.float32)
            + bc1_ref[...], 0.0)
        g = jnp.maximum(
            jnp.dot(g, wc2_ref[...], preferred_element_type=jnp.float32)
            + bc2_ref[...], 0.0)
        out_ref[...] = (jnp.dot(g, wc3_ref[...],
                                preferred_element_type=jnp.float32)
                        + bc3_ref[...])


def _row_spec():
    return pl.BlockSpec((RB, D), lambda i: (i, 0))


def _col_spec():
    return pl.BlockSpec((RB, 1), lambda i: (i, 0))


def _full_spec(r, c):
    return pl.BlockSpec((r, c), lambda i: (0, 0))


def kernel(x, edge_index, batch, W1, b1, W2, b2, Wc1, bc1, Wc2, bc2, Wc3, bc3):
    src = edge_index[0].astype(jnp.int32)
    dst = edge_index[1].astype(jnp.int32)
    src2 = jnp.concatenate(
        [src, jnp.zeros((EPAD - E,), jnp.int32)]).reshape(ROWS_E, EK)
    dst2 = jnp.concatenate(
        [dst, jnp.full((EPAD - E,), N, jnp.int32)]).reshape(ROWS_E, EK)
    batch2 = batch.astype(jnp.int32).reshape(N, 1)

    zeros_r = jnp.zeros((16, D), jnp.float32)

    dego, lsrc, ldst, cnts = _route_call(src2, dst2)
    d = dego.reshape(NW, DEGS)[:, :NPT].reshape(NW * NPT)[:N].reshape(N, 1)

    h1, h1p, dinv = pl.pallas_call(
        _mm1_kernel,
        grid=(GRID,),
        in_specs=[_row_spec(), _full_spec(D, D), _col_spec()],
        out_specs=(_row_spec(), _row_spec(), _col_spec()),
        out_shape=(
            jax.ShapeDtypeStruct((N, D), jnp.float32),
            jax.ShapeDtypeStruct((N, D), jnp.float32),
            jax.ShapeDtypeStruct((N, 1), jnp.float32),
        ),
    )(x, W1, d)

    aggo1 = _agg_call(h1p, lsrc, ldst, cnts, zeros_r)
    a1 = aggo1[:N]

    h2, h2p = pl.pallas_call(
        _mm2_kernel,
        grid=(GRID,),
        in_specs=[_row_spec(), _row_spec(), _col_spec(),
                  _full_spec(1, D), _full_spec(D, D)],
        out_specs=(_row_spec(), _row_spec()),
        out_shape=(
            jax.ShapeDtypeStruct((N, D), jnp.float32),
            jax.ShapeDtypeStruct((N, D), jnp.float32),
        ),
    )(a1, h1, dinv, b1.reshape(1, D), W2)

    aggo2 = _agg_call(h2p, lsrc, ldst, cnts, zeros_r)
    a2 = aggo2[:N]

    out = pl.pallas_call(
        _final_kernel,
        grid=(GRID,),
        in_specs=[_row_spec(), _row_spec(), _col_spec(),
                  _full_spec(1, D), _col_spec(),
                  _full_spec(D, 128), _full_spec(1, 128),
                  _full_spec(128, 64), _full_spec(1, 64),
                  _full_spec(64, 1), _full_spec(1, 1)],
        out_specs=pl.BlockSpec((G, 1), lambda i: (0, 0)),
        out_shape=jax.ShapeDtypeStruct((G, 1), jnp.float32),
        scratch_shapes=[
            pltpu.VMEM((G, D), jnp.float32),
            pltpu.VMEM((G, 1), jnp.float32),
        ],
    )(a2, h2, dinv, b2.reshape(1, D), batch2,
      Wc1, bc1.reshape(1, 128), Wc2, bc2.reshape(1, 64),
      Wc3, bc3.reshape(1, 1))
    return out


# speedup vs baseline: 1.1779x; 1.0702x over previous
import jax
import jax.numpy as jnp
from jax import lax
from jax.experimental import pallas as pl
from jax.experimental.pallas import tpu as pltpu
from jax.experimental.pallas import tpu_sc as plsc

N = 10000
E = 160000
D = 256
G = 32

NC, NS, L = 2, 16, 16
NW = NC * NS
EK = 128
EPAD = 163840
ROWS_E = EPAD // EK
NPT = 320
ACC_R = 336
DUMP = 320
CAP = 8192
GK = 64
SUP = 32
DEGS = 384
CL = 128

RB = 2000
GRID = N // RB

_mesh = plsc.VectorSubcoreMesh(
    core_axis_name="c", subcore_axis_name="s", num_cores=NC, num_subcores=NS
)

_SCAN_SUP = 64
_N_SCAN = ROWS_E // _SCAN_SUP


def _route_body(src_hbm, dst_hbm, deg_hbm, lsrc_hbm, ldst_hbm, cnt_hbm,
                src_sv, dst_sv, lsrc_v, ldst_v, hist_v, cnt_v):
    c = lax.axis_index("c")
    s = lax.axis_index("s")
    w = c * NS + s
    base = w * NPT

    zero16f = jnp.zeros((L,), jnp.float32)
    zero16i = jnp.zeros((L,), jnp.int32)
    dump16 = jnp.full((L,), DUMP, jnp.int32)
    ones16 = jnp.ones((L,), jnp.float32)

    @pl.loop(0, DEGS // L)
    def _zero_hist(u):
        hist_v[pl.ds(u * L, L)] = zero16f

    @pl.loop(0, CAP // L)
    def _prefill(u):
        lsrc_v[pl.ds(u * L, L)] = zero16i
        ldst_v[pl.ds(u * L, L)] = dump16

    def _scan_sup(t, cnt):
        pltpu.sync_copy(src_hbm.at[pl.ds(t * _SCAN_SUP, _SCAN_SUP)], src_sv)
        pltpu.sync_copy(dst_hbm.at[pl.ds(t * _SCAN_SUP, _SCAN_SUP)], dst_sv)

        def _group(u, cnt):
            r = u // (EK // L)
            col = (u % (EK // L)) * L
            d = dst_sv[r, pl.ds(col, L)]
            sv = src_sv[r, pl.ds(col, L)]
            own = lax.shift_right_logical(d * 52429, 24)
            lidx = d - base
            m = own == w
            safe = cnt < CAP - 2 * L
            msafe = m & safe
            plsc.addupdate_scatter(hist_v, [lidx], ones16, mask=msafe)
            plsc.store_compressed(lsrc_v.at[pl.ds(cnt, L)], sv, mask=msafe)
            plsc.store_compressed(ldst_v.at[pl.ds(cnt, L)], lidx, mask=msafe)
            npop = plsc.all_reduce_population_count(msafe)
            return cnt + (npop[0] if npop.ndim else npop)

        return lax.fori_loop(0, _SCAN_SUP * EK // L, _group, cnt)

    cnt = lax.fori_loop(0, _N_SCAN, _scan_sup, jnp.int32(0))

    cnt16 = jnp.broadcast_to(cnt, (L,)).astype(jnp.int32)

    @pl.loop(0, CL // L)
    def _fill_cnt(u):
        cnt_v[pl.ds(u * L, L)] = cnt16

    pltpu.sync_copy(hist_v, deg_hbm.at[pl.ds(w * DEGS, DEGS)])
    pltpu.sync_copy(lsrc_v, lsrc_hbm.at[pl.ds(w * CAP, CAP)])
    pltpu.sync_copy(ldst_v, ldst_hbm.at[pl.ds(w * CAP, CAP)])
    pltpu.sync_copy(cnt_v, cnt_hbm.at[pl.ds(w * CL, CL)])


_route_call = pl.kernel(
    _route_body,
    out_type=(
        jax.ShapeDtypeStruct((NW * DEGS,), jnp.float32),
        jax.ShapeDtypeStruct((NW * CAP,), jnp.int32),
        jax.ShapeDtypeStruct((NW * CAP,), jnp.int32),
        jax.ShapeDtypeStruct((NW * CL,), jnp.int32),
    ),
    mesh=_mesh,
    compiler_params=pltpu.CompilerParams(needs_layout_passes=False),
    scratch_types=[
        pltpu.VMEM((_SCAN_SUP, EK), jnp.int32),
        pltpu.VMEM((_SCAN_SUP, EK), jnp.int32),
        pltpu.VMEM((CAP,), jnp.int32),
        pltpu.VMEM((CAP,), jnp.int32),
        pltpu.VMEM((DEGS,), jnp.float32),
        pltpu.VMEM((CL,), jnp.int32),
    ],
)


def _agg_body(h_hbm, lsrc_hbm, ldst_hbm, cnt_hbm, zeros_hbm, out_hbm,
              lsrc_v, ldst_v, cnt_v, rows0, rows1, sem0, sem1, acc):
    c = lax.axis_index("c")
    s = lax.axis_index("s")
    w = c * NS + s

    pltpu.sync_copy(cnt_hbm.at[pl.ds(w * CL, CL)], cnt_v)
    cnt = cnt_v[pl.ds(0, L)][0]
    n_ch = lax.div(cnt + (GK - 1), GK)
    n_sup = lax.div(n_ch + (SUP - 1), SUP)

    @pl.loop(0, ACC_R // 112)
    def _zero(k):
        pltpu.sync_copy(zeros_hbm, acc.at[pl.ds(k * 112, 112)])

    bufs = (rows0, rows1)
    sems = (sem0, sem1)

    def _super(t, _):
        pltpu.sync_copy(lsrc_hbm.at[pl.ds(w * CAP + t * (SUP * GK), SUP * GK)],
                        lsrc_v)
        pltpu.sync_copy(ldst_hbm.at[pl.ds(w * CAP + t * (SUP * GK), SUP * GK)],
                        ldst_v)
        ch0 = t * SUP

        @pl.when(ch0 + 0 < n_ch)
        def _():
            pltpu.async_copy(h_hbm.at[lsrc_v.at[pl.ds(0, GK)]], rows0, sem0)

        @pl.when(ch0 + 1 < n_ch)
        def _():
            pltpu.async_copy(h_hbm.at[lsrc_v.at[pl.ds(GK, GK)]], rows1, sem1)

        @pl.loop(0, SUP, step=2)
        def _chunk(k):
            for b in range(2):
                kk = k + b
                buf = bufs[b]
                sem = sems[b]

                @pl.when(ch0 + kk < n_ch)
                def _():
                    pltpu.make_async_copy(
                        h_hbm.at[lsrc_v.at[pl.ds(kk * GK, GK)]], buf,
                        sem).wait()

                    @pl.loop(0, GK // L)
                    def _edges(gidx):
                        ld16 = ldst_v[pl.ds(kk * GK + gidx * L, L)]
                        prev_ld = None
                        prev_vals = None
                        for e2 in range(L):
                            e = gidx * L + e2
                            vals = [buf[e, pl.ds(seg * L, L)]
                                    for seg in range(D // L)]
                            if prev_vals is not None:
                                for seg in range(D // L):
                                    plsc.addupdate(
                                        acc.at[prev_ld, pl.ds(seg * L, L)],
                                        prev_vals[seg])
                            prev_ld = ld16[e2]
                            prev_vals = vals
                        for seg in range(D // L):
                            plsc.addupdate(
                                acc.at[prev_ld, pl.ds(seg * L, L)],
                                prev_vals[seg])

                    @pl.when((kk + 2 < SUP) & (ch0 + kk + 2 < n_ch))
                    def _():
                        pltpu.async_copy(
                            h_hbm.at[lsrc_v.at[pl.ds((kk + 2) * GK, GK)]],
                            buf, sem)

        return ()

    lax.fori_loop(0, n_sup, _super, ())

    @pl.loop(0, NPT // 64)
    def _writeout(k):
        pltpu.sync_copy(acc.at[pl.ds(k * 64, 64)],
                        out_hbm.at[pl.ds(w * NPT + k * 64, 64)])


_agg_call = pl.kernel(
    _agg_body,
    out_type=jax.ShapeDtypeStruct((NW * NPT, D), jnp.float32),
    mesh=_mesh,
    compiler_params=pltpu.CompilerParams(needs_layout_passes=False),
    scratch_types=[
        pltpu.VMEM((SUP * GK,), jnp.int32),
        pltpu.VMEM((SUP * GK,), jnp.int32),
        pltpu.VMEM((CL,), jnp.int32),
        pltpu.VMEM((GK, D), jnp.float32),
        pltpu.VMEM((GK, D), jnp.float32),
        pltpu.SemaphoreType.DMA,
        pltpu.SemaphoreType.DMA,
        pltpu.VMEM((ACC_R, D), jnp.float32),
    ],
)


def _mm1_kernel(x_ref, w_ref, d_ref, h_ref, hp_ref, dinv_ref):
    dinv = lax.rsqrt(d_ref[...] + 1.0)
    h = jnp.dot(x_ref[...], w_ref[...], preferred_element_type=jnp.float32)
    h_ref[...] = h
    hp_ref[...] = h * dinv
    dinv_ref[...] = dinv


def _mm2_kernel(a_ref, h1_ref, dinv_ref, b_ref, w_ref, h2_ref, h2p_ref):
    di = dinv_ref[...]
    z = jnp.maximum(di * a_ref[...] + di * di * h1_ref[...] + b_ref[...], 0.0)
    h2 = jnp.dot(z, w_ref[...], preferred_element_type=jnp.float32)
    h2_ref[...] = h2
    h2p_ref[...] = h2 * di


def _final_kernel(a_ref, h2_ref, dinv_ref, b_ref, batch_ref,
                  wc1_ref, bc1_ref, wc2_ref, bc2_ref, wc3_ref, bc3_ref,
                  out_ref, gacc, cacc):
    i = pl.program_id(0)

    @pl.when(i == 0)
    def _():
        gacc[...] = jnp.zeros((G, D), jnp.float32)
        cacc[...] = jnp.zeros((G, 1), jnp.float32)

    di = dinv_ref[...]
    z = jnp.maximum(di * a_ref[...] + di * di * h2_ref[...] + b_ref[...], 0.0)
    bvec = batch_ref[...]
    oh = (bvec == lax.broadcasted_iota(jnp.int32, (RB, G), 1)).astype(
        jnp.float32)
    gacc[...] += lax.dot_general(oh, z, (((0,), (0,)), ((), ())),
                                 preferred_element_type=jnp.float32,
                                 precision=lax.Precision.HIGHEST)
    cacc[...] += lax.dot_general(oh, jnp.ones((RB, 1), jnp.float32),
                                 (((0,), (0,)), ((), ())),
                                 preferred_element_type=jnp.float32,
                                 precision=lax.Precision.HIGHEST)

    @pl.when(i == GRID - 1)
    def _():
        g = gacc[...] / jnp.maximum(cacc[...], 1.0)
        g = jnp.maximum(
            jnp.dot(g, wc1_ref[...], preferred_element_type=jnp.float32)
            + bc1_ref[...], 0.0)
        g = jnp.maximum(
            jnp.dot(g, wc2_ref[...], preferred_element_type=jnp.float32)
            + bc2_ref[...], 0.0)
        out_ref[...] = (jnp.dot(g, wc3_ref[...],
                                preferred_element_type=jnp.float32)
                        + bc3_ref[...])


def _row_spec():
    return pl.BlockSpec((RB, D), lambda i: (i, 0))


def _col_spec():
    return pl.BlockSpec((RB, 1), lambda i: (i, 0))


def _full_spec(r, c):
    return pl.BlockSpec((r, c), lambda i: (0, 0))


def kernel(x, edge_index, batch, W1, b1, W2, b2, Wc1, bc1, Wc2, bc2, Wc3, bc3):
    src = edge_index[0].astype(jnp.int32)
    dst = edge_index[1].astype(jnp.int32)
    src2 = jnp.concatenate(
        [src, jnp.zeros((EPAD - E,), jnp.int32)]).reshape(ROWS_E, EK)
    dst2 = jnp.concatenate(
        [dst, jnp.full((EPAD - E,), N, jnp.int32)]).reshape(ROWS_E, EK)
    batch2 = batch.astype(jnp.int32).reshape(N, 1)

    zeros_r = jnp.zeros((112, D), jnp.float32)

    dego, lsrc, ldst, cnts = _route_call(src2, dst2)
    d = dego.reshape(NW, DEGS)[:, :NPT].reshape(NW * NPT)[:N].reshape(N, 1)

    h1, h1p, dinv = pl.pallas_call(
        _mm1_kernel,
        grid=(GRID,),
        in_specs=[_row_spec(), _full_spec(D, D), _col_spec()],
        out_specs=(_row_spec(), _row_spec(), _col_spec()),
        out_shape=(
            jax.ShapeDtypeStruct((N, D), jnp.float32),
            jax.ShapeDtypeStruct((N, D), jnp.float32),
            jax.ShapeDtypeStruct((N, 1), jnp.float32),
        ),
    )(x, W1, d)

    aggo1 = _agg_call(h1p, lsrc, ldst, cnts, zeros_r)
    a1 = aggo1[:N]

    h2, h2p = pl.pallas_call(
        _mm2_kernel,
        grid=(GRID,),
        in_specs=[_row_spec(), _row_spec(), _col_spec(),
                  _full_spec(1, D), _full_spec(D, D)],
        out_specs=(_row_spec(), _row_spec()),
        out_shape=(
            jax.ShapeDtypeStruct((N, D), jnp.float32),
            jax.ShapeDtypeStruct((N, D), jnp.float32),
        ),
    )(a1, h1, dinv, b1.reshape(1, D), W2)

    aggo2 = _agg_call(h2p, lsrc, ldst, cnts, zeros_r)
    a2 = aggo2[:N]

    out = pl.pallas_call(
        _final_kernel,
        grid=(GRID,),
        in_specs=[_row_spec(), _row_spec(), _col_spec(),
                  _full_spec(1, D), _col_spec(),
                  _full_spec(D, 128), _full_spec(1, 128),
                  _full_spec(128, 64), _full_spec(1, 64),
                  _full_spec(64, 1), _full_spec(1, 1)],
        out_specs=pl.BlockSpec((G, 1), lambda i: (0, 0)),
        out_shape=jax.ShapeDtypeStruct((G, 1), jnp.float32),
        scratch_shapes=[
            pltpu.VMEM((G, D), jnp.float32),
            pltpu.VMEM((G, 1), jnp.float32),
        ],
    )(a2, h2, dinv, b2.reshape(1, D), batch2,
      Wc1, bc1.reshape(1, 128), Wc2, bc2.reshape(1, 64),
      Wc3, bc3.reshape(1, 1))
    return out
